# baseline (speedup 1.0000x reference)
import functools

import jax
import jax.numpy as jnp
from jax import lax
from jax.experimental import pallas as pl
from jax.experimental.pallas import tpu as pltpu
from jax.experimental.pallas import tpu_sc as plsc

_N = 10000
_DIM = 128
_NC = 2
_NS = 16
_NW = _NC * _NS
_CHUNK = 128
_LANES = 16
_NBUF = 2

_ROWS_BASE = 624
_ROWS_LAST = _N - 15 * _ROWS_BASE


def _sc_aggregate(x, row, col, vals):
    e_pad = row.shape[0]
    ept = e_pad // _NW
    n_chunks = ept // _CHUNK

    mesh = plsc.VectorSubcoreMesh(core_axis_name="c", subcore_axis_name="s")

    @functools.partial(
        pl.kernel,
        out_type=jax.ShapeDtypeStruct((_NC, _N, _DIM), jnp.float32),
        mesh=mesh,
        scratch_types=[
            [pltpu.VMEM((_CHUNK,), jnp.int32)] * _NBUF,
            [pltpu.VMEM((_CHUNK,), jnp.int32)] * _NBUF,
            [pltpu.VMEM((_CHUNK,), jnp.float32)] * _NBUF,
            [pltpu.VMEM((_CHUNK, _DIM), jnp.float32)] * _NBUF,
            pltpu.VMEM_SHARED((_N, _DIM), jnp.float32),
            [pltpu.SemaphoreType.DMA] * _NBUF,
            [pltpu.SemaphoreType.DMA] * _NBUF,
            [pltpu.SemaphoreType.DMA] * _NBUF,
            [pltpu.SemaphoreType.DMA] * _NBUF,
            [pltpu.SemaphoreType.DMA] * _NBUF,
        ],
    )
    def agg(x_hbm, row_hbm, col_hbm, vals_hbm, out_hbm,
            cidx, ridx, vv, bufs, h_sh, csems, rsems, vsems, gsems, ssems):
        cid = lax.axis_index("c")
        sid = lax.axis_index("s")
        wid = sid * _NC + cid

        def _zero_row(i, carry):
            for j in range(_DIM // _LANES):
                bufs[0][i, pl.ds(j * _LANES, _LANES)] = jnp.zeros(
                    (_LANES,), jnp.float32)
            return carry
        lax.fori_loop(0, _CHUNK, _zero_row, 0)
        base_row = sid * _ROWS_BASE
        for cpy in range(_ROWS_BASE // _CHUNK):
            pltpu.sync_copy(bufs[0], h_sh.at[pl.ds(base_row + cpy * _CHUNK, _CHUNK)])
        rem = _ROWS_BASE - (_ROWS_BASE // _CHUNK) * _CHUNK

        @pl.when(sid < _NS - 1)
        def _zero_tail_base():
            pltpu.sync_copy(bufs[0].at[pl.ds(0, rem)],
                            h_sh.at[pl.ds(base_row + _ROWS_BASE - rem, rem)])

        @pl.when(sid == _NS - 1)
        def _zero_tail_last():
            pltpu.sync_copy(bufs[0], h_sh.at[pl.ds(base_row + _ROWS_BASE - rem, _CHUNK)])
        plsc.subcore_barrier()

        def _cidx_start(c, s):
            base = wid * ept + c * _CHUNK
            pltpu.async_copy(col_hbm.at[pl.ds(base, _CHUNK)], cidx[s], csems[s])

        def _cidx_wait(c, s):
            base = wid * ept + c * _CHUNK
            pltpu.make_async_copy(col_hbm.at[pl.ds(base, _CHUNK)], cidx[s],
                                  csems[s]).wait()

        def _ridx_start(c, s):
            base = wid * ept + c * _CHUNK
            pltpu.async_copy(row_hbm.at[pl.ds(base, _CHUNK)], ridx[s], rsems[s])

        def _ridx_wait(c, s):
            base = wid * ept + c * _CHUNK
            pltpu.make_async_copy(row_hbm.at[pl.ds(base, _CHUNK)], ridx[s],
                                  rsems[s]).wait()

        def _vv_start(c, s):
            base = wid * ept + c * _CHUNK
            pltpu.async_copy(vals_hbm.at[pl.ds(base, _CHUNK)], vv[s], vsems[s])

        def _vv_wait(c, s):
            base = wid * ept + c * _CHUNK
            pltpu.make_async_copy(vals_hbm.at[pl.ds(base, _CHUNK)], vv[s],
                                  vsems[s]).wait()

        def _gather_start(b):
            pltpu.async_copy(x_hbm.at[cidx[b]], bufs[b], gsems[b])

        def _gather_wait(b):
            pltpu.make_async_copy(x_hbm.at[cidx[b]], bufs[b], gsems[b]).wait()

        def _scatter_start(b):
            pltpu.async_copy(bufs[b], h_sh.at[ridx[b]], ssems[b], add=True)

        def _scatter_wait(b):
            pltpu.make_async_copy(bufs[b], h_sh.at[ridx[b]], ssems[b]).wait()

        def _scale(b):
            def grp(g, inner):
                val16 = vv[b][pl.ds(g * _LANES, _LANES)]
                for e in range(_LANES):
                    bc = val16[e]
                    r = g * _LANES + e
                    for j in range(_DIM // _LANES):
                        sl = pl.ds(j * _LANES, _LANES)
                        bufs[b][r, sl] = bufs[b][r, sl] * bc
                return inner
            lax.fori_loop(0, _CHUNK // _LANES, grp, 0)

        for b in range(_NBUF):
            _cidx_start(b, b)
            _ridx_start(b, b)
            _vv_start(b, b)
        for b in range(_NBUF):
            _cidx_wait(b, b)
            _gather_start(b)

        n_iter = n_chunks // _NBUF

        def body(it, carry):
            c0 = it * _NBUF
            more = it < n_iter - 1
            for k in range(_NBUF):
                _gather_wait(k)

                @pl.when(more)
                def _prefetch_cidx():
                    _cidx_start(c0 + k + _NBUF, k)
                _vv_wait(c0 + k, k)
                _scale(k)

                @pl.when(more)
                def _prefetch_vv():
                    _vv_start(c0 + k + _NBUF, k)
                _ridx_wait(c0 + k, k)
                _scatter_start(k)
            for k in range(_NBUF):
                _scatter_wait(k)

                @pl.when(more)
                def _next_gather():
                    _ridx_start(c0 + k + _NBUF, k)
                    _cidx_wait(c0 + k + _NBUF, k)
                    _gather_start(k)
            return carry
        lax.fori_loop(0, n_iter, body, 0)

        plsc.subcore_barrier()

        @pl.when(sid < _NS - 1)
        def _write_base():
            pltpu.sync_copy(h_sh.at[pl.ds(base_row, _ROWS_BASE)],
                            out_hbm.at[cid, pl.ds(base_row, _ROWS_BASE)])

        @pl.when(sid == _NS - 1)
        def _write_last():
            pltpu.sync_copy(h_sh.at[pl.ds(base_row, _ROWS_LAST)],
                            out_hbm.at[cid, pl.ds(base_row, _ROWS_LAST)])

    return agg(x, row, col, vals)


_BN = 1000


def _gate_body(hp_ref, x_ref, w_ref, b_ref, o_ref):
    h = hp_ref[0] + hp_ref[1]
    z = jnp.sum(h * w_ref[...], axis=1, keepdims=True) + b_ref[0, 0]
    g = jax.nn.sigmoid(z)
    o_ref[...] = g * h + (1.0 - g) * x_ref[...]


def _gate(hp, x, W_gate, b_gate):
    wt = W_gate.reshape(1, _DIM)
    bb = b_gate.reshape(1, 1)
    grid = _N // _BN
    return pl.pallas_call(
        _gate_body,
        grid=(grid,),
        in_specs=[
            pl.BlockSpec((_NC, _BN, _DIM), lambda i: (0, i, 0)),
            pl.BlockSpec((_BN, _DIM), lambda i: (i, 0)),
            pl.BlockSpec((1, _DIM), lambda i: (0, 0)),
            pl.BlockSpec(memory_space=pltpu.SMEM),
        ],
        out_specs=pl.BlockSpec((_BN, _DIM), lambda i: (i, 0)),
        out_shape=jax.ShapeDtypeStruct((_N, _DIM), jnp.float32),
    )(hp, x, wt, bb)


def kernel(x, adj_indices, adj_values, W_gate, b_gate):
    row = adj_indices[0].astype(jnp.int32)
    col = adj_indices[1].astype(jnp.int32)
    vals = adj_values.astype(jnp.float32)
    e = row.shape[0]
    unit = _NW * _CHUNK * _NBUF
    e_pad = ((e + unit - 1) // unit) * unit
    pad = e_pad - e
    if pad:
        row = jnp.concatenate([row, jnp.zeros((pad,), jnp.int32)])
        col = jnp.concatenate([col, jnp.zeros((pad,), jnp.int32)])
        vals = jnp.concatenate([vals, jnp.zeros((pad,), jnp.float32)])
    hp = _sc_aggregate(x, row, col, vals)
    return _gate(hp, x, W_gate, b_gate)
